# baseline (speedup 1.0000x reference)
import functools

import jax
import jax.numpy as jnp
from jax import lax
from jax.experimental import pallas as pl
from jax.experimental.pallas import tpu as pltpu
from jax.experimental.pallas import tpu_sc as plsc

_GAMMA = 12.0
_HIDDEN = 64
_LANES = 16


@functools.lru_cache(maxsize=None)
def _build(batch, nc, ns):
    nw = nc * ns
    bpw = batch // nw
    mesh = plsc.VectorSubcoreMesh(core_axis_name="c", subcore_axis_name="s")

    @functools.partial(
        pl.kernel,
        mesh=mesh,
        out_type=jax.ShapeDtypeStruct((batch,), jnp.float32),
        compiler_params=pltpu.CompilerParams(
            needs_layout_passes=False, use_tc_tiling_on_sc=False
        ),
        scratch_types=[
            pltpu.VMEM((bpw,), jnp.int32),
            pltpu.VMEM((bpw,), jnp.int32),
            pltpu.VMEM((bpw,), jnp.int32),
            pltpu.VMEM((bpw, _HIDDEN), jnp.float32),
            pltpu.VMEM((bpw, _HIDDEN), jnp.float32),
            pltpu.VMEM((bpw, _HIDDEN), jnp.float32),
            pltpu.VMEM((bpw,), jnp.float32),
            pltpu.SemaphoreType.DMA,
        ],
    )
    def kge_score(hidx_hbm, ridx_hbm, tidx_hbm, ent_hbm, rel_hbm, out_hbm,
                  hidx, ridx, tidx, hrows, rrows, trows, outv, sem):
        wid = lax.axis_index("s") * nc + lax.axis_index("c")
        base = wid * bpw
        pltpu.sync_copy(hidx_hbm.at[pl.ds(base, bpw)], hidx)
        pltpu.sync_copy(ridx_hbm.at[pl.ds(base, bpw)], ridx)
        pltpu.sync_copy(tidx_hbm.at[pl.ds(base, bpw)], tidx)
        cph = pltpu.async_copy(ent_hbm.at[hidx], hrows, sem)
        cpr = pltpu.async_copy(rel_hbm.at[ridx], rrows, sem)
        cpt = pltpu.async_copy(ent_hbm.at[tidx], trows, sem)
        cph.wait()
        cpr.wait()
        cpt.wait()

        lanes = lax.iota(jnp.int32, _LANES)

        def body(g, carry):
            vec = jnp.zeros((_LANES,), jnp.float32)
            for l in range(_LANES):
                i = g * _LANES + l
                acc = None
                for j in range(_HIDDEN // _LANES):
                    h = hrows[i, pl.ds(j * _LANES, _LANES)]
                    r = rrows[i, pl.ds(j * _LANES, _LANES)]
                    t = trows[i, pl.ds(j * _LANES, _LANES)]
                    x = jnp.abs(h + r - t)
                    acc = x if acc is None else acc + x
                vec = jnp.where(lanes == l, _GAMMA - jnp.sum(acc), vec)
            outv[pl.ds(g * _LANES, _LANES)] = vec
            return carry

        lax.fori_loop(0, bpw // _LANES, body, 0)
        pltpu.sync_copy(outv, out_hbm.at[pl.ds(base, bpw)])

    return kge_score


def kernel(sample, entity_embedding, relation_embedding):
    batch = sample.shape[0]
    info = plsc.get_sparse_core_info()
    sample = sample.astype(jnp.int32)
    heads = sample[:, 0]
    rels = sample[:, 1]
    tails = sample[:, 2]
    fn = _build(batch, info.num_cores, info.num_subcores)
    out = fn(heads, rels, tails, entity_embedding, relation_embedding)
    return out[:, None]
